# baseline (speedup 1.0000x reference)
import functools

import jax
import jax.numpy as jnp
from jax import lax
from jax.experimental import pallas as pl
from jax.experimental.pallas import tpu as pltpu
from jax.experimental.pallas import tpu_sc as plsc

_NUM_TILES = 16
_L = 16
_LN2_OVER_2P23 = 0.6931471805599453 / (1 << 23)
_ONE_BITS = 0x3F800000


def _log_newton(s):
  bits = lax.bitcast_convert_type(s, jnp.int32)
  logv = (bits - _ONE_BITS).astype(jnp.float32) * _LN2_OVER_2P23
  for _ in range(2):
    logv = logv - 1.0 + s * jnp.exp(-logv)
  return logv


def _tree_reduce(vals, op):
  vals = list(vals)
  while len(vals) > 1:
    nxt = [op(vals[i], vals[i + 1]) for i in range(0, len(vals) - 1, 2)]
    if len(vals) % 2:
      nxt.append(vals[-1])
    vals = nxt
  return vals[0]


def _depth_loss_sc(C, H, W, M, bin_w, res_w, pb_flat, pr_flat, mi, mf, out,
                   mi_v, mf_v, bidx, ridx, brow, rrow, part, allp, outv,
                   shared, sem, sem2):
  tid = lax.axis_index("s")
  n_groups = M // _L
  lane = jnp.arange(_L, dtype=jnp.int32)

  with jax.named_scope("ph_meta"):
    cm = pltpu.async_copy(mi.at[tid], mi_v, sem)
    cf = pltpu.async_copy(mf.at[tid], mf_v, sem2)
    cm.wait()

  base_off = tid * (C * H * W)
  off0s = []
  tbvs = []
  for g in range(n_groups):
    o = g * _L
    yv = jnp.clip(mi_v[pl.ds(0 * M + o, _L)], 0, H - 1)
    xv = jnp.clip(mi_v[pl.ds(1 * M + o, _L)], 0, W - 1)
    tbvs.append(mi_v[pl.ds(2 * M + o, _L)])
    off0s.append(base_off + yv * W + xv)

  copies = []
  for c in range(C):
    for g in range(n_groups):
      bidx[pl.ds(c * M + g * _L, _L)] = off0s[g] + c * (H * W)
    copies.append(pltpu.async_copy(pb_flat.at[bidx.at[pl.ds(c * M, M)]],
                                   brow.at[pl.ds(c * M, M)], sem))
  for g in range(n_groups):
    ridx[pl.ds(g * _L, _L)] = off0s[g] + tbvs[g] * (H * W)
  copies.append(pltpu.async_copy(pr_flat.at[ridx], rrow, sem))
  with jax.named_scope("ph_gather"):
    cf.wait()
    for cp in copies:
      cp.wait()

  ces = []
  l1s = []
  mks = []
  for g in range(n_groups):
    o = g * _L
    vs = [brow[pl.ds(c * M + o, _L)] for c in range(C)]
    m = _tree_reduce(vs, jnp.maximum)
    s = _tree_reduce([jnp.exp(v - m) for v in vs], jnp.add)
    lse = m + _log_newton(s)
    tbv = tbvs[g]
    vt = jnp.zeros((_L,), jnp.float32)
    for c in range(C):
      vt = jnp.where(tbv == c, vs[c], vt)
    mk = mf_v[pl.ds(1 * M + o, _L)]
    ces.append((lse - vt) * mk)
    l1s.append(jnp.abs(rrow[pl.ds(o, _L)] - mf_v[pl.ds(0 * M + o, _L)]) * mk)
    mks.append(mk)

  with jax.named_scope("ph_compute_done"):
    pass
  part[pl.ds(0 * _L, _L)] = _tree_reduce(ces, jnp.add)
  part[pl.ds(1 * _L, _L)] = _tree_reduce(l1s, jnp.add)
  part[pl.ds(2 * _L, _L)] = _tree_reduce(mks, jnp.add)
  with jax.named_scope("ph_stage"):
    pltpu.sync_copy(part, shared.at[pl.ds(tid * 3 * _L, 3 * _L)])
    plsc.subcore_barrier()

  @pl.when(tid == 0)
  def _finish():
    pltpu.sync_copy(shared, allp)
    tot_ce = allp[pl.ds(0, _L)]
    tot_l1 = allp[pl.ds(_L, _L)]
    tot_mk = allp[pl.ds(2 * _L, _L)]
    for w in range(1, _NUM_TILES):
      tot_ce = tot_ce + allp[pl.ds((w * 3 + 0) * _L, _L)]
      tot_l1 = tot_l1 + allp[pl.ds((w * 3 + 1) * _L, _L)]
      tot_mk = tot_mk + allp[pl.ds((w * 3 + 2) * _L, _L)]
    ce_v = jnp.full((_L,), jnp.sum(tot_ce), jnp.float32)
    l1_v = jnp.full((_L,), jnp.sum(tot_l1), jnp.float32)
    denom = jnp.maximum(jnp.full((_L,), jnp.sum(tot_mk), jnp.float32), 1.0)
    bin_loss = ce_v / denom
    res_loss = l1_v / denom
    total = bin_w * bin_loss + res_w * res_loss
    sel = jnp.where(lane == 0, bin_loss,
                    jnp.where(lane == 1, res_loss, total))
    outv[...] = sel
    pltpu.sync_copy(outv, out)


def kernel(pred_bins, pred_residuals, target_bins, target_residuals, indices,
           mask):
  B, C, H, W = pred_bins.shape
  M = target_bins.shape[1]
  mi = jnp.stack(
      [indices[..., 0].astype(jnp.int32),
       indices[..., 1].astype(jnp.int32),
       target_bins.astype(jnp.int32)], axis=1).reshape(B, 3 * M)
  mf = jnp.stack([target_residuals, mask], axis=1).reshape(B, 2 * M)
  pb_flat = pred_bins.reshape(-1)
  pr_flat = pred_residuals.reshape(-1)

  mesh = plsc.VectorSubcoreMesh(core_axis_name="c", subcore_axis_name="s",
                                num_cores=1)
  body = functools.partial(_depth_loss_sc, C, H, W, M, 1.0, 0.1)
  out = pl.kernel(
      body,
      out_type=jax.ShapeDtypeStruct((_L,), jnp.float32),
      mesh=mesh,
      compiler_params=pltpu.CompilerParams(needs_layout_passes=False),
      scratch_types=[
          pltpu.VMEM((3 * M,), jnp.int32),
          pltpu.VMEM((2 * M,), jnp.float32),
          pltpu.VMEM((C * M,), jnp.int32),
          pltpu.VMEM((M,), jnp.int32),
          pltpu.VMEM((C * M,), jnp.float32),
          pltpu.VMEM((M,), jnp.float32),
          pltpu.VMEM((3 * _L,), jnp.float32),
          pltpu.VMEM((_NUM_TILES * 3 * _L,), jnp.float32),
          pltpu.VMEM((_L,), jnp.float32),
          pltpu.VMEM_SHARED((_NUM_TILES * 3 * _L,), jnp.float32),
          pltpu.SemaphoreType.DMA,
          pltpu.SemaphoreType.DMA,
      ],
  )(pb_flat, pr_flat, mi, mf)
  return (out[0], out[1], out[2])


# speedup vs baseline: 1.0015x; 1.0015x over previous
import functools

import jax
import jax.numpy as jnp
from jax import lax
from jax.experimental import pallas as pl
from jax.experimental.pallas import tpu as pltpu
from jax.experimental.pallas import tpu_sc as plsc

_NUM_TILES = 16
_L = 16
_LN2_OVER_2P23 = 0.6931471805599453 / (1 << 23)
_ONE_BITS = 0x3F800000


def _log_newton(s):
  bits = lax.bitcast_convert_type(s, jnp.int32)
  logv = (bits - _ONE_BITS).astype(jnp.float32) * _LN2_OVER_2P23
  for _ in range(2):
    logv = logv - 1.0 + s * jnp.exp(-logv)
  return logv


def _tree_reduce(vals, op):
  vals = list(vals)
  while len(vals) > 1:
    nxt = [op(vals[i], vals[i + 1]) for i in range(0, len(vals) - 1, 2)]
    if len(vals) % 2:
      nxt.append(vals[-1])
    vals = nxt
  return vals[0]


def _depth_loss_sc(C, H, W, M, bin_w, res_w, pb_flat, pr_flat, mi, mf, out,
                   mi_v, mf_v, bidx, ridx, brow, rrow, part, allp, outv,
                   shared, sem, sem2):
  tid = lax.axis_index("s")
  n_groups = M // _L
  lane = jnp.arange(_L, dtype=jnp.int32)

  cm = pltpu.async_copy(mi.at[tid], mi_v, sem)
  cf = pltpu.async_copy(mf.at[tid], mf_v, sem2)
  cm.wait()

  base_off = tid * (C * H * W)
  off0s = []
  tbvs = []
  for g in range(n_groups):
    o = g * _L
    yv = jnp.clip(mi_v[pl.ds(0 * M + o, _L)], 0, H - 1)
    xv = jnp.clip(mi_v[pl.ds(1 * M + o, _L)], 0, W - 1)
    tbvs.append(mi_v[pl.ds(2 * M + o, _L)])
    off0s.append(base_off + yv * W + xv)

  copies = []
  for c in range(C):
    for g in range(n_groups):
      bidx[pl.ds(c * M + g * _L, _L)] = off0s[g] + c * (H * W)
    copies.append(pltpu.async_copy(pb_flat.at[bidx.at[pl.ds(c * M, M)]],
                                   brow.at[pl.ds(c * M, M)], sem))
  for g in range(n_groups):
    ridx[pl.ds(g * _L, _L)] = off0s[g] + tbvs[g] * (H * W)
  copies.append(pltpu.async_copy(pr_flat.at[ridx], rrow, sem))
  cf.wait()
  for cp in copies:
    cp.wait()

  ces = []
  l1s = []
  mks = []
  for g in range(n_groups):
    o = g * _L
    vs = [brow[pl.ds(c * M + o, _L)] for c in range(C)]
    m = _tree_reduce(vs, jnp.maximum)
    s = _tree_reduce([jnp.exp(v - m) for v in vs], jnp.add)
    lse = m + _log_newton(s)
    tbv = tbvs[g]
    vt = jnp.zeros((_L,), jnp.float32)
    for c in range(C):
      vt = jnp.where(tbv == c, vs[c], vt)
    mk = mf_v[pl.ds(1 * M + o, _L)]
    ces.append((lse - vt) * mk)
    l1s.append(jnp.abs(rrow[pl.ds(o, _L)] - mf_v[pl.ds(0 * M + o, _L)]) * mk)
    mks.append(mk)

  part[pl.ds(0 * _L, _L)] = _tree_reduce(ces, jnp.add)
  part[pl.ds(1 * _L, _L)] = _tree_reduce(l1s, jnp.add)
  part[pl.ds(2 * _L, _L)] = _tree_reduce(mks, jnp.add)
  pltpu.sync_copy(part, shared.at[pl.ds(tid * 3 * _L, 3 * _L)])
  plsc.subcore_barrier()

  @pl.when(tid == 0)
  def _finish():
    pltpu.sync_copy(shared, allp)
    tot_ce = allp[pl.ds(0, _L)]
    tot_l1 = allp[pl.ds(_L, _L)]
    tot_mk = allp[pl.ds(2 * _L, _L)]
    for w in range(1, _NUM_TILES):
      tot_ce = tot_ce + allp[pl.ds((w * 3 + 0) * _L, _L)]
      tot_l1 = tot_l1 + allp[pl.ds((w * 3 + 1) * _L, _L)]
      tot_mk = tot_mk + allp[pl.ds((w * 3 + 2) * _L, _L)]
    ce_v = jnp.full((_L,), jnp.sum(tot_ce), jnp.float32)
    l1_v = jnp.full((_L,), jnp.sum(tot_l1), jnp.float32)
    denom = jnp.maximum(jnp.full((_L,), jnp.sum(tot_mk), jnp.float32), 1.0)
    bin_loss = ce_v / denom
    res_loss = l1_v / denom
    total = bin_w * bin_loss + res_w * res_loss
    sel = jnp.where(lane == 0, bin_loss,
                    jnp.where(lane == 1, res_loss, total))
    outv[...] = sel
    pltpu.sync_copy(outv, out)


def kernel(pred_bins, pred_residuals, target_bins, target_residuals, indices,
           mask):
  B, C, H, W = pred_bins.shape
  M = target_bins.shape[1]
  mi = jnp.stack(
      [indices[..., 0].astype(jnp.int32),
       indices[..., 1].astype(jnp.int32),
       target_bins.astype(jnp.int32)], axis=1).reshape(B, 3 * M)
  mf = jnp.stack([target_residuals, mask], axis=1).reshape(B, 2 * M)
  pb_flat = pred_bins.reshape(-1)
  pr_flat = pred_residuals.reshape(-1)

  mesh = plsc.VectorSubcoreMesh(core_axis_name="c", subcore_axis_name="s",
                                num_cores=1)
  body = functools.partial(_depth_loss_sc, C, H, W, M, 1.0, 0.1)
  out = pl.kernel(
      body,
      out_type=jax.ShapeDtypeStruct((_L,), jnp.float32),
      mesh=mesh,
      compiler_params=pltpu.CompilerParams(needs_layout_passes=False),
      scratch_types=[
          pltpu.VMEM((3 * M,), jnp.int32),
          pltpu.VMEM((2 * M,), jnp.float32),
          pltpu.VMEM((C * M,), jnp.int32),
          pltpu.VMEM((M,), jnp.int32),
          pltpu.VMEM((C * M,), jnp.float32),
          pltpu.VMEM((M,), jnp.float32),
          pltpu.VMEM((3 * _L,), jnp.float32),
          pltpu.VMEM((_NUM_TILES * 3 * _L,), jnp.float32),
          pltpu.VMEM((_L,), jnp.float32),
          pltpu.VMEM_SHARED((_NUM_TILES * 3 * _L,), jnp.float32),
          pltpu.SemaphoreType.DMA,
          pltpu.SemaphoreType.DMA,
      ],
  )(pb_flat, pr_flat, mi, mf)
  return (out[0], out[1], out[2])


# speedup vs baseline: 1.0073x; 1.0058x over previous
import functools

import jax
import jax.numpy as jnp
from jax import lax
from jax.experimental import pallas as pl
from jax.experimental.pallas import tpu as pltpu
from jax.experimental.pallas import tpu_sc as plsc

_NUM_TILES = 16
_L = 16
_LN2_OVER_2P23 = 0.6931471805599453 / (1 << 23)
_ONE_BITS = 0x3F800000


def _log_newton(s):
  bits = lax.bitcast_convert_type(s, jnp.int32)
  logv = (bits - _ONE_BITS).astype(jnp.float32) * _LN2_OVER_2P23
  for _ in range(2):
    logv = logv - 1.0 + s * jnp.exp(-logv)
  return logv


def _tree_reduce(vals, op):
  vals = list(vals)
  while len(vals) > 1:
    nxt = [op(vals[i], vals[i + 1]) for i in range(0, len(vals) - 1, 2)]
    if len(vals) % 2:
      nxt.append(vals[-1])
    vals = nxt
  return vals[0]


def _depth_loss_sc(C, H, W, M, bin_w, res_w, pb_flat, pr_flat, mi, mf, out,
                   mi_v, mf_v, bidx, ridx, brow, rrow, part, allp, outv,
                   shared, sem, sem2):
  tid = lax.axis_index("s")
  n_groups = M // _L
  lane = jnp.arange(_L, dtype=jnp.int32)

  cm = pltpu.async_copy(mi.at[tid], mi_v, sem)
  cf = pltpu.async_copy(mf.at[tid], mf_v, sem2)
  cm.wait()

  base_off = tid * (C * H * W)
  off0s = []
  tbvs = []
  for g in range(n_groups):
    o = g * _L
    yv = jnp.clip(mi_v[pl.ds(0 * M + o, _L)], 0, H - 1)
    xv = jnp.clip(mi_v[pl.ds(1 * M + o, _L)], 0, W - 1)
    tbvs.append(jnp.clip(mi_v[pl.ds(2 * M + o, _L)], 0, C - 1))
    off0s.append(base_off + yv * W + xv)

  copies = []
  for c in range(C):
    for g in range(n_groups):
      bidx[pl.ds(c * M + g * _L, _L)] = off0s[g] + c * (H * W)
    copies.append(pltpu.async_copy(pb_flat.at[bidx.at[pl.ds(c * M, M)]],
                                   brow.at[pl.ds(c * M, M)], sem))
  for g in range(n_groups):
    ridx[pl.ds(g * _L, _L)] = off0s[g] + tbvs[g] * (H * W)
  copies.append(pltpu.async_copy(pr_flat.at[ridx], rrow, sem))
  cf.wait()
  for cp in copies:
    cp.wait()

  ces = []
  l1s = []
  mks = []
  for g in range(n_groups):
    o = g * _L
    vs = [brow[pl.ds(c * M + o, _L)] for c in range(C)]
    m = _tree_reduce(vs, jnp.maximum)
    s = _tree_reduce([jnp.exp(v - m) for v in vs], jnp.add)
    lse = m + _log_newton(s)
    tbv = tbvs[g]
    vt = jnp.zeros((_L,), jnp.float32)
    for c in range(C):
      vt = jnp.where(tbv == c, vs[c], vt)
    mk = mf_v[pl.ds(1 * M + o, _L)]
    ces.append((lse - vt) * mk)
    l1s.append(jnp.abs(rrow[pl.ds(o, _L)] - mf_v[pl.ds(0 * M + o, _L)]) * mk)
    mks.append(mk)

  part[pl.ds(0 * _L, _L)] = _tree_reduce(ces, jnp.add)
  part[pl.ds(1 * _L, _L)] = _tree_reduce(l1s, jnp.add)
  part[pl.ds(2 * _L, _L)] = _tree_reduce(mks, jnp.add)
  pltpu.sync_copy(part, shared.at[pl.ds(tid * 3 * _L, 3 * _L)])
  plsc.subcore_barrier()

  @pl.when(tid == 0)
  def _finish():
    pltpu.sync_copy(shared, allp)
    tot_ce = allp[pl.ds(0, _L)]
    tot_l1 = allp[pl.ds(_L, _L)]
    tot_mk = allp[pl.ds(2 * _L, _L)]
    for w in range(1, _NUM_TILES):
      tot_ce = tot_ce + allp[pl.ds((w * 3 + 0) * _L, _L)]
      tot_l1 = tot_l1 + allp[pl.ds((w * 3 + 1) * _L, _L)]
      tot_mk = tot_mk + allp[pl.ds((w * 3 + 2) * _L, _L)]
    ce_v = jnp.full((_L,), jnp.sum(tot_ce), jnp.float32)
    l1_v = jnp.full((_L,), jnp.sum(tot_l1), jnp.float32)
    denom = jnp.maximum(jnp.full((_L,), jnp.sum(tot_mk), jnp.float32), 1.0)
    bin_loss = ce_v / denom
    res_loss = l1_v / denom
    total = bin_w * bin_loss + res_w * res_loss
    sel = jnp.where(lane == 0, bin_loss,
                    jnp.where(lane == 1, res_loss, total))
    outv[...] = sel
    pltpu.sync_copy(outv, out)


def kernel(pred_bins, pred_residuals, target_bins, target_residuals, indices,
           mask):
  B, C, H, W = pred_bins.shape
  M = target_bins.shape[1]
  mi = jnp.stack(
      [indices[..., 0].astype(jnp.int32),
       indices[..., 1].astype(jnp.int32),
       target_bins.astype(jnp.int32)], axis=1).reshape(B, 3 * M)
  mf = jnp.stack([target_residuals, mask], axis=1).reshape(B, 2 * M)
  pb_flat = pred_bins.reshape(-1)
  pr_flat = pred_residuals.reshape(-1)

  mesh = plsc.VectorSubcoreMesh(core_axis_name="c", subcore_axis_name="s",
                                num_cores=1)
  body = functools.partial(_depth_loss_sc, C, H, W, M, 1.0, 0.1)
  out = pl.kernel(
      body,
      out_type=jax.ShapeDtypeStruct((_L,), jnp.float32),
      mesh=mesh,
      compiler_params=pltpu.CompilerParams(needs_layout_passes=False),
      scratch_types=[
          pltpu.VMEM((3 * M,), jnp.int32),
          pltpu.VMEM((2 * M,), jnp.float32),
          pltpu.VMEM((C * M,), jnp.int32),
          pltpu.VMEM((M,), jnp.int32),
          pltpu.VMEM((C * M,), jnp.float32),
          pltpu.VMEM((M,), jnp.float32),
          pltpu.VMEM((3 * _L,), jnp.float32),
          pltpu.VMEM((_NUM_TILES * 3 * _L,), jnp.float32),
          pltpu.VMEM((_L,), jnp.float32),
          pltpu.VMEM_SHARED((_NUM_TILES * 3 * _L,), jnp.float32),
          pltpu.SemaphoreType.DMA,
          pltpu.SemaphoreType.DMA,
      ],
  )(pb_flat, pr_flat, mi, mf)
  return (out[0], out[1], out[2])
